# baseline (speedup 1.0000x reference)
import jax
import jax.numpy as jnp
from jax.experimental import pallas as pl
from jax.experimental.pallas import tpu as pltpu

_DROP_WIDTH = 64
_STRIPES_NUM = 2

_NB = 6
_D = 3
_CHB = 8


def _stripe_params(B, total_width):
    key = jax.random.key(42)
    k_dist, k_bgn = jax.random.split(key)
    distances = jax.random.randint(k_dist, (B, _STRIPES_NUM), 0, _DROP_WIDTH)
    u = jax.random.uniform(k_bgn, (B, _STRIPES_NUM))
    bgns = jnp.floor(u * (total_width - distances).astype(jnp.float32)).astype(
        jnp.int32
    )
    ends = bgns + distances.astype(jnp.int32)
    return jnp.concatenate([bgns, ends], axis=1)


def _body(params_ref, x_hbm, o_hbm, bufs, sin, sout):
    B, W, C = x_hbm.shape
    nchk = B // _CHB

    def in_copy(i):
        k = i % _NB
        return pltpu.make_async_copy(
            x_hbm.at[pl.ds(i * _CHB, _CHB)], bufs.at[k], sin.at[k]
        )

    def out_copy(i):
        k = i % _NB
        return pltpu.make_async_copy(
            bufs.at[k], o_hbm.at[pl.ds(i * _CHB, _CHB)], sout.at[k]
        )

    iota = jax.lax.broadcasted_iota(jnp.int32, (_DROP_WIDTH, 1), 0)

    def fixup(j):
        k = j % _NB
        for i in range(_CHB):
            b = j * _CHB + i
            for s in range(_STRIPES_NUM):
                st = jnp.minimum(params_ref[b, s], W - _DROP_WIDTH)
                idx = iota + st
                drop = (idx >= params_ref[b, 0]) & (
                    idx < params_ref[b, _STRIPES_NUM]
                )
                for t in range(1, _STRIPES_NUM):
                    drop |= (idx >= params_ref[b, t]) & (
                        idx < params_ref[b, _STRIPES_NUM + t]
                    )
                keep = jnp.where(drop, 0.0, 1.0)
                win = pl.ds(st, _DROP_WIDTH)
                bufs[k, i, win, :] = bufs[k, i, win, :] * keep

    for i in range(nchk + _D):
        if i < nchk:
            if i >= _NB:
                out_copy(i - _NB).wait()
            in_copy(i).start()
        j = i - _D
        if 0 <= j < nchk:
            in_copy(j).wait()
            fixup(j)
            out_copy(j).start()
    for j in range(nchk - _NB, nchk):
        out_copy(j).wait()


@jax.jit
def kernel(input):
    B, W, C = input.shape
    params = _stripe_params(B, W)
    return pl.pallas_call(
        _body,
        in_specs=[
            pl.BlockSpec(memory_space=pltpu.SMEM),
            pl.BlockSpec(memory_space=pltpu.HBM),
        ],
        out_specs=pl.BlockSpec(memory_space=pltpu.HBM),
        out_shape=jax.ShapeDtypeStruct((B, W, C), input.dtype),
        scratch_shapes=[
            pltpu.VMEM((_NB, _CHB, W, C), jnp.float32),
            pltpu.SemaphoreType.DMA((_NB,)),
            pltpu.SemaphoreType.DMA((_NB,)),
        ],
    )(params, input)
